# baseline (speedup 1.0000x reference)
import functools

import jax
import jax.numpy as jnp
import numpy as np
from jax.experimental import pallas as pl
from jax.experimental.pallas import tpu as pltpu
from jax.sharding import Mesh, PartitionSpec as P


def _rms(x, g, eps):
    ms = jnp.mean(jnp.square(x), axis=-1, keepdims=True)
    return x * jax.lax.rsqrt(ms + eps) * g


def _prep_kernel(wq_ref, wk_ref, wv_ref, wo_ref, w1_ref, w2_ref,
                 oq_ref, ok_ref, ov_ref, oo_ref, o1_ref, o2_ref, *, scale):
    oq_ref[...] = (wq_ref[...] * scale).astype(jnp.bfloat16)
    ok_ref[...] = wk_ref[...].astype(jnp.bfloat16)
    ov_ref[...] = wv_ref[...].astype(jnp.bfloat16)
    oo_ref[...] = wo_ref[...].astype(jnp.bfloat16)
    o1_ref[...] = w1_ref[...].astype(jnp.bfloat16)
    o2_ref[...] = w2_ref[...].astype(jnp.bfloat16)


def _prep_weights(wq, wk, wv, wo, w1, w2, scale):
    D, HID = w1.shape
    n = 32
    sq = pl.BlockSpec((D // n, D), lambda i: (i, 0))
    s1 = pl.BlockSpec((D // n, HID), lambda i: (i, 0))
    s2 = pl.BlockSpec((HID // n, D), lambda i: (i, 0))
    return pl.pallas_call(
        functools.partial(_prep_kernel, scale=scale),
        out_shape=[
            jax.ShapeDtypeStruct((D, D), jnp.bfloat16),
            jax.ShapeDtypeStruct((D, D), jnp.bfloat16),
            jax.ShapeDtypeStruct((D, D), jnp.bfloat16),
            jax.ShapeDtypeStruct((D, D), jnp.bfloat16),
            jax.ShapeDtypeStruct((D, HID), jnp.bfloat16),
            jax.ShapeDtypeStruct((HID, D), jnp.bfloat16),
        ],
        grid=(n,),
        in_specs=[sq, sq, sq, sq, s1, s2],
        out_specs=[sq, sq, sq, sq, s1, s2],
        compiler_params=pltpu.CompilerParams(
            dimension_semantics=("parallel",),
        ),
    )(wq, wk, wv, wo, w1, w2)


def _fused_kernel(
    lat_ref, emb_ref, mask_ref,
    g1_ref, wq_ref, bq_ref, wk_ref, bk_ref, wv_ref, bv_ref, wo_ref, bo_ref,
    g2_ref, w1_ref, b1_ref, w2_ref, b2_ref,
    out_ref,
    *, num_heads, q_scale,
):
    eps = jnp.float32(jnp.finfo(jnp.float32).eps)
    H = num_heads
    D = lat_ref.shape[-1]
    hd = D // H

    x = lat_ref[...]
    xn = _rms(x, g1_ref[...], eps)
    xn_b = xn.astype(jnp.bfloat16)

    e = emb_ref[...].astype(jnp.bfloat16)
    kf = jnp.dot(e, wk_ref[...], preferred_element_type=jnp.float32) + bk_ref[...]
    vf = jnp.dot(e, wv_ref[...], preferred_element_type=jnp.float32) + bv_ref[...]
    kb = kf.astype(jnp.bfloat16)
    vb = vf.astype(jnp.bfloat16)

    q = (jnp.dot(xn_b, wq_ref[...], preferred_element_type=jnp.float32)
         + bq_ref[...] * jnp.float32(q_scale))
    qb = q.astype(jnp.bfloat16)

    mask = mask_ref[...]
    bias = jnp.where(mask > 0, jnp.float32(0.0), jnp.float32(-1e30))

    ctx_t_parts = []
    for h in range(H):
        sl = slice(h * hd, (h + 1) * hd)
        s = jax.lax.dot_general(
            qb[:, sl], kb[:, sl],
            (((1,), (1,)), ((), ())),
            preferred_element_type=jnp.float32)
        s = s + bias
        m = jnp.max(s, axis=-1, keepdims=True)
        p = jnp.exp(s - m)
        denom = jnp.sum(p, axis=-1, keepdims=True)
        recip = pl.reciprocal(denom, approx=True).reshape(1, -1)
        ctx_t = jax.lax.dot_general(
            vb[:, sl], p.astype(jnp.bfloat16),
            (((0,), (1,)), ((), ())),
            preferred_element_type=jnp.float32)
        ctx_t_parts.append((ctx_t * recip).astype(jnp.bfloat16))
    ctx_t = jnp.concatenate(ctx_t_parts, axis=0)

    attn = jax.lax.dot_general(
        ctx_t, wo_ref[...],
        (((0,), (0,)), ((), ())),
        preferred_element_type=jnp.float32) + bo_ref[...]
    x1 = attn + xn

    x2 = _rms(x1, g2_ref[...], eps)
    h1 = jnp.dot(x2.astype(jnp.bfloat16), w1_ref[...],
                 preferred_element_type=jnp.float32) + b1_ref[...]
    h1 = h1 * jax.nn.sigmoid(h1)
    ff = jnp.dot(h1.astype(jnp.bfloat16), w2_ref[...],
                 preferred_element_type=jnp.float32) + b2_ref[...]

    out_ref[...] = (ff + x2).astype(out_ref.dtype)


def _impl(latents, embeddings, mask, g1, wq, bq, wk, bk, wv, bv, wo, bo,
          g2, w1, b1, w2, b2):
    B, L, D = latents.shape
    _, S, _ = embeddings.shape
    H = 8
    hd = D // H
    scale = 1.0 / float(hd) ** 0.5

    def row(v):
        return jnp.asarray(v).reshape(1, -1).astype(jnp.float32)

    wq_b, wk_b, wv_b, wo_b, w1_b, w2_b = _prep_weights(
        wq, wk, wv, wo, w1, w2, scale)

    weights = [
        row(g1),
        wq_b, row(bq),
        wk_b, row(bk),
        wv_b, row(bv),
        wo_b, row(bo),
        row(g2),
        w1_b, row(b1),
        w2_b, row(b2),
    ]

    buffered = getattr(pl, "Buffered", None)

    def build(single_buffer_weights):
        wkw = {"pipeline_mode": buffered(1)} if single_buffer_weights else {}

        def full_spec(arr):
            nd = arr.ndim
            return pl.BlockSpec(arr.shape, lambda b, _nd=nd: (0,) * _nd, **wkw)

        in_specs = [
            pl.BlockSpec((None, L, D), lambda b: (b, 0, 0)),
            pl.BlockSpec((None, S, D), lambda b: (b, 0, 0)),
            pl.BlockSpec((None, 1, S), lambda b: (b, 0, 0)),
        ] + [full_spec(w) for w in weights]

        return pl.pallas_call(
            functools.partial(_fused_kernel, num_heads=H, q_scale=scale),
            out_shape=jax.ShapeDtypeStruct((B, L, D), latents.dtype),
            grid=(B,),
            in_specs=in_specs,
            out_specs=pl.BlockSpec((None, L, D), lambda b: (b, 0, 0)),
            compiler_params=pltpu.CompilerParams(
                dimension_semantics=("parallel",),
                vmem_limit_bytes=(64 * 1024 * 1024 * 7) // 8,
            ),
        )

    args = (latents.astype(jnp.float32), embeddings.astype(jnp.float32),
            mask, *weights)
    if buffered is not None:
        try:
            return build(True)(*args)
        except Exception:
            return build(False)(*args)
    return build(False)(*args)


def kernel(latents, embeddings, mask, g1, wq, bq, wk, bk, wv, bv, wo, bo,
           g2, w1, b1, w2, b2):
    B, _, _ = latents.shape
    _, S, _ = embeddings.shape

    if mask is None:
        mask = jnp.ones((B, S), dtype=jnp.float32)
    mask = mask.astype(jnp.float32).reshape(B, 1, S)

    devs = jax.devices()
    if len(devs) >= 2 and B % 2 == 0:
        mesh = Mesh(np.asarray(devs[:2]), ("x",))
        sharded = jax.shard_map(
            _impl,
            mesh=mesh,
            in_specs=(P("x"), P("x"), P("x")) + (P(),) * 14,
            out_specs=P("x"),
            check_vma=False,
        )
        return sharded(latents, embeddings, mask, g1, wq, bq, wk, bk,
                       wv, bv, wo, bo, g2, w1, b1, w2, b2)
    return _impl(latents, embeddings, mask, g1, wq, bq, wk, bk,
                 wv, bv, wo, bo, g2, w1, b1, w2, b2)


# speedup vs baseline: 2.8341x; 2.8341x over previous
import functools

import jax
import jax.numpy as jnp
from jax.experimental import pallas as pl
from jax.experimental.pallas import tpu as pltpu


def _rms(x, g, eps):
    ms = jnp.mean(jnp.square(x), axis=-1, keepdims=True)
    return x * jax.lax.rsqrt(ms + eps) * g


def _fused_kernel(
    lat_ref, emb_ref, mask_ref,
    g1_ref, wq_hbm, bq_ref, wk_hbm, bk_ref, wv_hbm, bv_ref, wo_hbm, bo_ref,
    g2_ref, w1_hbm, b1_ref, w2_hbm, b2_ref,
    out_ref,
    wq_s, wk_s, wv_s, wo_s, w1_s, w2_s, st_a, st_b, sem_a, sem_b,
    *, num_heads, q_scale,
):
    eps = jnp.float32(jnp.finfo(jnp.float32).eps)
    H = num_heads
    D = lat_ref.shape[-1]
    hd = D // H
    HID = w1_s.shape[-1]

    ch_a = st_a.shape[1]
    ch_b = st_b.shape[1]
    chunks_a = []
    for src, dst in ((wk_hbm, wk_s), (wv_hbm, wv_s), (wq_hbm, wq_s),
                     (wo_hbm, wo_s), (w2_hbm, w2_s)):
        for r in range(0, src.shape[0], ch_a):
            chunks_a.append((src, r, dst))
    chunks_b = [(w1_hbm, r, w1_s) for r in range(0, D, ch_b)]

    @pl.when(pl.program_id(0) == 0)
    def _load_weights():
        def copy_a(i, slot):
            src, r, _ = chunks_a[i]
            return pltpu.make_async_copy(
                src.at[pl.ds(r, ch_a), :], st_a.at[slot], sem_a.at[slot])

        def copy_b(i, slot):
            src, r, _ = chunks_b[i]
            return pltpu.make_async_copy(
                src.at[pl.ds(r, ch_b), :], st_b.at[slot], sem_b.at[slot])

        copy_a(0, 0).start()
        for i in range(len(chunks_a)):
            if i + 1 < len(chunks_a):
                copy_a(i + 1, (i + 1) % 2).start()
            elif chunks_b:
                copy_b(0, 0).start()
            copy_a(i, i % 2).wait()
            src, r, dst = chunks_a[i]
            val = st_a[i % 2]
            if dst is wq_s:
                val = val * jnp.float32(q_scale)
            dst[pl.ds(r, ch_a), :] = val.astype(jnp.bfloat16)
        for i in range(len(chunks_b)):
            if i + 1 < len(chunks_b):
                copy_b(i + 1, (i + 1) % 2).start()
            copy_b(i, i % 2).wait()
            _, r, dst = chunks_b[i]
            dst[pl.ds(r, ch_b), :] = st_b[i % 2].astype(jnp.bfloat16)

    x = lat_ref[...]
    xn = _rms(x, g1_ref[...], eps)
    xn_b = xn.astype(jnp.bfloat16)

    e = emb_ref[...].astype(jnp.bfloat16)
    kf = jnp.dot(e, wk_s[...], preferred_element_type=jnp.float32) + bk_ref[...]
    vf = jnp.dot(e, wv_s[...], preferred_element_type=jnp.float32) + bv_ref[...]
    kb = kf.astype(jnp.bfloat16)
    vb = vf.astype(jnp.bfloat16)

    q = (jnp.dot(xn_b, wq_s[...], preferred_element_type=jnp.float32)
         + bq_ref[...] * jnp.float32(q_scale))
    qb = q.astype(jnp.bfloat16)

    mask = mask_ref[...]
    bias = jnp.where(mask > 0, jnp.float32(0.0), jnp.float32(-1e30))

    ctx_t_parts = []
    for h in range(H):
        sl = slice(h * hd, (h + 1) * hd)
        s = jax.lax.dot_general(
            qb[:, sl], kb[:, sl],
            (((1,), (1,)), ((), ())),
            preferred_element_type=jnp.float32)
        s = s + bias
        m = jnp.max(s, axis=-1, keepdims=True)
        p = jnp.exp(s - m)
        denom = jnp.sum(p, axis=-1, keepdims=True)
        recip = pl.reciprocal(denom, approx=True).reshape(1, -1)
        ctx_t = jax.lax.dot_general(
            vb[:, sl], p.astype(jnp.bfloat16),
            (((0,), (1,)), ((), ())),
            preferred_element_type=jnp.float32)
        ctx_t_parts.append((ctx_t * recip).astype(jnp.bfloat16))
    ctx_t = jnp.concatenate(ctx_t_parts, axis=0)

    attn = jax.lax.dot_general(
        ctx_t, wo_s[...],
        (((0,), (0,)), ((), ())),
        preferred_element_type=jnp.float32) + bo_ref[...]
    x1 = attn + xn

    x2 = _rms(x1, g2_ref[...], eps)
    h1 = jnp.dot(x2.astype(jnp.bfloat16), w1_s[...],
                 preferred_element_type=jnp.float32) + b1_ref[...]
    h1 = h1 * jax.nn.sigmoid(h1)
    ff = jnp.dot(h1.astype(jnp.bfloat16), w2_s[...],
                 preferred_element_type=jnp.float32) + b2_ref[...]

    out_ref[...] = (ff + x2).astype(out_ref.dtype)


def kernel(latents, embeddings, mask, g1, wq, bq, wk, bk, wv, bv, wo, bo,
           g2, w1, b1, w2, b2):
    B, L, D = latents.shape
    _, S, _ = embeddings.shape
    H = 8
    hd = D // H
    HID = w1.shape[-1]
    scale = 1.0 / float(hd) ** 0.5

    if mask is None:
        mask = jnp.ones((B, S), dtype=jnp.float32)
    mask = mask.astype(jnp.float32).reshape(B, 1, S)

    def row(v):
        return jnp.asarray(v).reshape(1, -1).astype(jnp.float32)

    f32 = jnp.float32
    params = [
        row(g1),
        wq.astype(f32), row(bq),
        wk.astype(f32), row(bk),
        wv.astype(f32), row(bv),
        wo.astype(f32), row(bo),
        row(g2),
        w1.astype(f32), row(b1),
        w2.astype(f32), row(b2),
    ]
    hbm_idx = {1, 3, 5, 7, 10, 12}

    buffered = getattr(pl, "Buffered", None)

    def build(single_buffer_weights):
        wkw = {"pipeline_mode": buffered(1)} if single_buffer_weights else {}

        def spec_for(i, arr):
            if i in hbm_idx:
                return pl.BlockSpec(memory_space=pl.ANY)
            nd = arr.ndim
            return pl.BlockSpec(arr.shape, lambda b, _nd=nd: (0,) * _nd, **wkw)

        in_specs = [
            pl.BlockSpec((None, L, D), lambda b: (b, 0, 0)),
            pl.BlockSpec((None, S, D), lambda b: (b, 0, 0)),
            pl.BlockSpec((None, 1, S), lambda b: (b, 0, 0)),
        ] + [spec_for(i, w) for i, w in enumerate(params)]

        return pl.pallas_call(
            functools.partial(_fused_kernel, num_heads=H, q_scale=scale),
            out_shape=jax.ShapeDtypeStruct((B, L, D), latents.dtype),
            grid=(B,),
            in_specs=in_specs,
            out_specs=pl.BlockSpec((None, L, D), lambda b: (b, 0, 0)),
            scratch_shapes=[
                pltpu.VMEM((D, D), jnp.bfloat16),
                pltpu.VMEM((D, D), jnp.bfloat16),
                pltpu.VMEM((D, D), jnp.bfloat16),
                pltpu.VMEM((D, D), jnp.bfloat16),
                pltpu.VMEM((D, HID), jnp.bfloat16),
                pltpu.VMEM((HID, D), jnp.bfloat16),
                pltpu.VMEM((2, min(512, D), D), jnp.float32),
                pltpu.VMEM((2, min(128, D), HID), jnp.float32),
                pltpu.SemaphoreType.DMA((2,)),
                pltpu.SemaphoreType.DMA((2,)),
            ],
            compiler_params=pltpu.CompilerParams(
                dimension_semantics=("arbitrary",),
                vmem_limit_bytes=(64 * 1024 * 1024 * 15) // 16,
            ),
        )

    args = (latents.astype(jnp.float32), embeddings.astype(jnp.float32),
            mask, *params)
    if buffered is not None:
        try:
            return build(True)(*args)
        except Exception:
            return build(False)(*args)
    return build(False)(*args)
